# Initial kernel scaffold; baseline (speedup 1.0000x reference)
import jax
import jax.numpy as jnp
from jax.experimental import pallas as pl


def kernel(features, positions, mask, W, b):
    raise NotImplementedError("write your pallas kernel here")


# speedup vs baseline: 1.1228x; 1.1228x over previous
import functools

import jax
import jax.numpy as jnp
from jax import lax
from jax.experimental import pallas as pl
from jax.experimental.pallas import tpu as pltpu
from jax.experimental.pallas import tpu_sc as plsc

B, N, D = 16, 4096, 256
K = 500
KP = 512
NV = N // 16


def _sc_body(sc_hbm, msk_hbm, posf_hbm, featf_hbm,
             feat_out, pos_out, probs_out,
             scv, mskv, keyv, ckv, ck2, cvv, cv2, histv, h32v, binv,
             posv, pov, prv, gidv, gid2v, rowsv, shared, sem):
    c = lax.axis_index("c")
    s = lax.axis_index("s")
    iot = lax.iota(jnp.int32, 16)
    zi = jnp.zeros((16,), jnp.int32)

    @pl.when(s < 8)
    def _sort():
        b = c * 8 + s
        pltpu.sync_copy(sc_hbm.at[b], scv)
        pltpu.sync_copy(msk_hbm.at[b], mskv)

        def zh(j, _):
            histv[pl.ds(j * 16, 16)] = zi
            return 0
        lax.fori_loop(0, 16, zh, 0)

        def hist_step(i, _):
            sf = scv[pl.ds(i * 16, 16)]
            mm = mskv[pl.ds(i * 16, 16)] != 0
            sf = jnp.where(mm, sf, -jnp.inf)
            bits = plsc.bitcast(sf, jnp.int32)
            k = jnp.where(bits < 0, ~bits, bits | jnp.int32(-2147483648))
            keyv[pl.ds(i * 16, 16)] = k
            di = (k >> 24) & 255
            cnt, lastm = plsc.scan_count(di)
            plsc.addupdate_scatter(histv, [di], cnt, mask=lastm)
            return 0
        lax.fori_loop(0, NV, hist_step, 0)

        def thr_step(j, carry):
            acc, t, cc, found = carry
            vi = 15 - j
            h = histv[pl.ds(vi * 16, 16)]
            hr = lax.rev(h, (0,))
            cs = plsc.cumsum(hr)
            gs = acc + cs
            m = gs >= K
            anyv = jnp.max(jnp.where(m, 1, 0))
            l0 = jnp.max(plsc.all_reduce_ffs(m))
            cand_c = jnp.max(jnp.where(iot == l0, gs, 0))
            cand_t = vi * 16 + 15 - l0
            take = (found == 0) & (anyv == 1)
            t = jnp.where(take, cand_t, t)
            cc = jnp.where(take, cand_c, cc)
            found = jnp.where(anyv == 1, 1, found)
            acc = acc + jnp.max(cs)
            return acc, t, cc, found
        _, t, cc, _ = lax.fori_loop(
            0, 16, thr_step,
            (jnp.int32(0), jnp.int32(0), jnp.int32(0), jnp.int32(0)))

        threshx = (t << 24) ^ jnp.int32(-2147483648)

        def comp_step(i, o):
            k = keyv[pl.ds(i * 16, 16)]
            m = (k ^ jnp.int32(-2147483648)) >= threshx
            mi = jnp.where(m, 1, 0)
            inc = plsc.cumsum(mi)
            addr = o + inc - 1
            plsc.store_scatter(ckv, [addr], k, mask=m)
            plsc.store_scatter(cvv, [addr], i * 16 + iot, mask=m)
            return o + jnp.sum(mi)
        csz = lax.fori_loop(0, NV, comp_step, jnp.int32(0))

        nv16 = ((csz + 15) // 16) * 16
        padm = iot < (nv16 - csz)
        plsc.store_scatter(ckv, [csz + iot], zi, mask=padm)
        plsc.store_scatter(cvv, [csz + iot], zi, mask=padm)
        nv = nv16 // 16

        bufs = [(ckv, cvv), (ck2, cv2)]
        for p in range(7):
            shift = 5 * p
            srck, srcv = bufs[p % 2]
            dstk, dstv = bufs[(p + 1) % 2]
            h32v[pl.ds(0, 16)] = zi
            h32v[pl.ds(16, 16)] = zi

            def ph1(i, _, srck=srck, shift=shift):
                k = srck[pl.ds(i * 16, 16)]
                dc = 31 - ((k >> shift) & 31)
                cnt, lastm = plsc.scan_count(dc)
                plsc.addupdate_scatter(h32v, [dc], cnt, mask=lastm)
                return 0
            lax.fori_loop(0, nv, ph1, 0)

            h0 = h32v[pl.ds(0, 16)]
            h1 = h32v[pl.ds(16, 16)]
            c0 = plsc.cumsum(h0)
            c1 = plsc.cumsum(h1)
            binv[pl.ds(0, 16)] = c0 - h0
            binv[pl.ds(16, 16)] = c1 - h1 + jnp.max(c0)

            def ph2(i, _, srck=srck, srcv=srcv, dstk=dstk, dstv=dstv,
                    shift=shift):
                k = srck[pl.ds(i * 16, 16)]
                v = srcv[pl.ds(i * 16, 16)]
                dc = 31 - ((k >> shift) & 31)
                cnt, lastm = plsc.scan_count(dc)
                base = plsc.load_gather(binv, [dc])
                addr = base + cnt - 1
                plsc.store_scatter(dstk, [addr], k)
                plsc.store_scatter(dstv, [addr], v)
                plsc.addupdate_scatter(binv, [dc], cnt, mask=lastm)
                return 0
            lax.fori_loop(0, nv, ph2, 0)

        def gid_step(i, _):
            idx = cv2[pl.ds(i * 16, 16)]
            gidv[pl.ds(i * 16, 16)] = idx + b * N
            return 0
        lax.fori_loop(0, KP // 16, gid_step, 0)
        pltpu.sync_copy(gidv, shared.at[s])

    plsc.subcore_barrier()

    @pl.when(s < 8)
    def _emit():
        b = c * 8 + s
        pltpu.sync_copy(posf_hbm.at[b], posv)

        def out_step(i, _):
            u = ck2[pl.ds(i * 16, 16)]
            bits = jnp.where(u < 0, u ^ jnp.int32(-2147483648), ~u)
            sc = plsc.bitcast(bits, jnp.float32)
            prv[pl.ds(i * 16, 16)] = 1.0 / (1.0 + jnp.exp(-sc))
            idx = cv2[pl.ds(i * 16, 16)]
            av = 2 * (i * 16 + iot)
            plsc.store_scatter(pov, [av], plsc.load_gather(posv, [2 * idx]))
            plsc.store_scatter(pov, [av + 1],
                               plsc.load_gather(posv, [2 * idx + 1]))
            return 0
        lax.fori_loop(0, KP // 16, out_step, 0)
        pltpu.sync_copy(prv, probs_out.at[b])
        pltpu.sync_copy(pov, pos_out.at[b])
        pltpu.async_copy(featf_hbm.at[gidv.at[pl.ds(0, 128)]],
                         rowsv.at[pl.ds(0, 128)], sem).wait()
        pltpu.async_copy(featf_hbm.at[gidv.at[pl.ds(128, 128)]],
                         rowsv.at[pl.ds(128, 128)], sem).wait()
        pltpu.sync_copy(rowsv, feat_out.at[b, pl.ds(0, 256)])

    @pl.when(s >= 8)
    def _gather2():
        b = c * 8 + (s - 8)
        pltpu.sync_copy(shared.at[s - 8], gid2v)
        pltpu.async_copy(featf_hbm.at[gid2v.at[pl.ds(256, 128)]],
                         rowsv.at[pl.ds(0, 128)], sem).wait()
        pltpu.async_copy(featf_hbm.at[gid2v.at[pl.ds(384, 128)]],
                         rowsv.at[pl.ds(128, 128)], sem).wait()
        pltpu.sync_copy(rowsv.at[pl.ds(0, 248)],
                        feat_out.at[b, pl.ds(256, 248)])


def _sc_select(scores, mask_i32, posf, featf):
    mesh = plsc.VectorSubcoreMesh(core_axis_name="c", subcore_axis_name="s")
    f = pl.kernel(
        _sc_body,
        out_type=(
            jax.ShapeDtypeStruct((B, 504, D), jnp.float32),
            jax.ShapeDtypeStruct((B, 2 * KP), jnp.float32),
            jax.ShapeDtypeStruct((B, KP), jnp.float32),
        ),
        mesh=mesh,
        scratch_types=[
            pltpu.VMEM((N,), jnp.float32),
            pltpu.VMEM((N,), jnp.int32),
            pltpu.VMEM((N,), jnp.int32),
            pltpu.VMEM((N + 16,), jnp.int32),
            pltpu.VMEM((N + 16,), jnp.int32),
            pltpu.VMEM((N + 16,), jnp.int32),
            pltpu.VMEM((N + 16,), jnp.int32),
            pltpu.VMEM((256,), jnp.int32),
            pltpu.VMEM((32,), jnp.int32),
            pltpu.VMEM((32,), jnp.int32),
            pltpu.VMEM((2 * N,), jnp.float32),
            pltpu.VMEM((2 * KP,), jnp.float32),
            pltpu.VMEM((KP,), jnp.float32),
            pltpu.VMEM((KP,), jnp.int32),
            pltpu.VMEM((KP,), jnp.int32),
            pltpu.VMEM((256, D), jnp.float32),
            pltpu.VMEM_SHARED((8, KP), jnp.int32),
            pltpu.SemaphoreType.DMA,
        ],
        compiler_params=pltpu.CompilerParams(needs_layout_passes=False),
    )
    return f(scores, mask_i32, posf, featf)


@jax.jit
def kernel(features, positions, mask, W, b):
    assert features.shape == (B, N, D)
    scores = (features @ W.T + b)[..., 0]
    featf = features.reshape(B * N, D)
    posf = positions.reshape(B, 2 * N)
    feat_out, pos_out, probs_out = _sc_select(scores, mask.astype(jnp.int32),
                                              posf, featf)
    pos = pos_out.reshape(B, KP, 2)[:, :K, :]
    return pos, feat_out[:, :K, :], probs_out[:, :K]
